# baseline (speedup 1.0000x reference)
import functools

import jax
import jax.numpy as jnp
from jax import lax
from jax.experimental import pallas as pl
from jax.experimental.pallas import tpu as pltpu
from jax.experimental.pallas import tpu_sc as plsc

VOCAB1 = 1000001
BATCH = 16384
EMBED_DIM = 16
PACK = 4
NUM_WORKERS = 32
B_PER_W = BATCH // NUM_WORKERS
CHUNK = 128
NCHUNK = B_PER_W // CHUNK
GROUPS = B_PER_W // 16

TBLK = 8192
NBLK = (VOCAB1 + TBLK - 1) // TBLK
QROWS = TBLK // PACK
PACKED_ROWS = NBLK * QROWS
PROW = PACK * EMBED_DIM


def _pack_body(x_ref, o_ref):
    eye = jnp.eye(EMBED_DIM, dtype=jnp.float32)
    acc = None
    for a in range(PACK):
        ea = jnp.pad(
            eye, ((0, 0), (a * EMBED_DIM, (PACK - 1 - a) * EMBED_DIM)))
        part = jax.lax.dot_general(
            x_ref[:, a * QROWS:(a + 1) * QROWS], ea, (((0,), (0,)), ((), ())),
            preferred_element_type=jnp.float32)
        acc = part if acc is None else acc + part
    o_ref[...] = acc


_pack_table = pl.pallas_call(
    _pack_body,
    grid=(NBLK,),
    in_specs=[pl.BlockSpec((EMBED_DIM, TBLK), lambda i: (0, i))],
    out_specs=pl.BlockSpec((QROWS, PROW), lambda i: (i, 0)),
    out_shape=jax.ShapeDtypeStruct((PACKED_ROWS, PROW), jnp.float32),
)

_mesh = plsc.VectorSubcoreMesh(core_axis_name="c", subcore_axis_name="s")

_QSHIFT = TBLK.bit_length() - 1
_QBITS = QROWS.bit_length() - 1
_AMASK = PACK - 1


@functools.partial(
    pl.kernel,
    mesh=_mesh,
    out_type=jax.ShapeDtypeStruct((BATCH,), jnp.float32),
    scratch_types=[
        pltpu.VMEM((B_PER_W, 2), jnp.int32),
        pltpu.VMEM((NCHUNK, CHUNK), jnp.int32),
        pltpu.VMEM((NCHUNK, CHUNK), jnp.int32),
        pltpu.VMEM((B_PER_W, PROW), jnp.float32),
        pltpu.VMEM((B_PER_W, PROW), jnp.float32),
        pltpu.VMEM((B_PER_W,), jnp.float32),
        pltpu.VMEM((1,), jnp.float32),
        pltpu.SemaphoreType.DMA,
        pltpu.SemaphoreType.DMA,
    ],
    compiler_params=pltpu.CompilerParams(
        needs_layout_passes=False, use_tc_tiling_on_sc=False),
)
def _mf_kernel(pairs_hbm, utab_hbm, itab_hbm, bias_hbm, out_hbm,
               pairs_v, uq_v, iq_v, urows_v, irows_v, out_v, bias_v,
               sem_u, sem_i):
    wid = lax.axis_index("s") * 2 + lax.axis_index("c")
    base = wid * B_PER_W

    pltpu.sync_copy(pairs_hbm.at[pl.ds(base, B_PER_W)], pairs_v)
    pltpu.sync_copy(bias_hbm, bias_v)

    iota = lax.iota(jnp.int32, 16)
    zeros16 = jnp.zeros((16,), jnp.int32)
    ones16 = jnp.ones((16,), jnp.int32)

    def _qid(vals):
        return jnp.bitwise_or(
            lax.shift_left(lax.shift_right_logical(vals, _QSHIFT), _QBITS),
            jnp.bitwise_and(vals, QROWS - 1))

    for g in range(GROUPS):
        rows = g * 16 + iota
        c, off = divmod(g * 16, CHUNK)
        uq_v[c, pl.ds(off, 16)] = _qid(plsc.load_gather(pairs_v, [rows, zeros16]))
        iq_v[c, pl.ds(off, 16)] = _qid(plsc.load_gather(pairs_v, [rows, ones16]))

    copies = []
    for c in range(NCHUNK):
        dst = pl.ds(c * CHUNK, CHUNK)
        copies.append(pltpu.make_async_copy(
            utab_hbm.at[uq_v.at[c]], urows_v.at[dst], sem_u))
        copies.append(pltpu.make_async_copy(
            itab_hbm.at[iq_v.at[c]], irows_v.at[dst], sem_i))
    for cp in copies:
        cp.start()
    for cp in copies:
        cp.wait()

    bias_vec = plsc.load_gather(bias_v, [zeros16])

    def _lane_base(vals):
        return lax.shift_left(
            jnp.bitwise_and(lax.shift_right_logical(vals, _QBITS), _AMASK), 4)

    def dot_body(g, carry):
        rows = g * 16 + iota
        sl = pl.ds(g * 16, 16)
        lbu = _lane_base(plsc.load_gather(pairs_v, [rows, zeros16]))
        lbi = _lane_base(plsc.load_gather(pairs_v, [rows, ones16]))
        acc = bias_vec
        for d in range(EMBED_DIM):
            u = plsc.load_gather(urows_v, [rows, lbu + d])
            v = plsc.load_gather(irows_v, [rows, lbi + d])
            acc = acc + u * v
        out_v[sl] = acc
        return carry

    lax.fori_loop(0, GROUPS, dot_body, 0)

    pltpu.sync_copy(out_v, out_hbm.at[pl.ds(base, B_PER_W)])


def kernel(sparse_inputs, user_table, item_table, bias):
    pairs = sparse_inputs.astype(jnp.int32)
    ut_p = _pack_table(user_table.T)
    it_p = _pack_table(item_table.T)
    return _mf_kernel(pairs, ut_p, it_p, bias)


# speedup vs baseline: 1.5078x; 1.5078x over previous
import functools

import jax
import jax.numpy as jnp
from jax import lax
from jax.experimental import pallas as pl
from jax.experimental.pallas import tpu as pltpu
from jax.experimental.pallas import tpu_sc as plsc

VOCAB1 = 1000001
BATCH = 16384
EMBED_DIM = 16
PACK = 8
NUM_WORKERS = 32
B_PER_W = BATCH // NUM_WORKERS
CHUNK = 128
NCHUNK = B_PER_W // CHUNK
GROUPS = B_PER_W // 16

TBLK = 8192
NBLK = (VOCAB1 + TBLK - 1) // TBLK
QROWS = TBLK // PACK
PACKED_ROWS = NBLK * QROWS
PROW = PACK * EMBED_DIM


def _pack_body(x_ref, o_ref):
    eye = jnp.eye(EMBED_DIM, dtype=jnp.float32)
    acc = None
    for a in range(PACK):
        ea = jnp.pad(
            eye, ((0, 0), (a * EMBED_DIM, (PACK - 1 - a) * EMBED_DIM)))
        part = jax.lax.dot_general(
            x_ref[:, a * QROWS:(a + 1) * QROWS], ea, (((0,), (0,)), ((), ())),
            preferred_element_type=jnp.float32)
        acc = part if acc is None else acc + part
    o_ref[...] = acc


_pack_table = pl.pallas_call(
    _pack_body,
    grid=(NBLK,),
    in_specs=[pl.BlockSpec((EMBED_DIM, TBLK), lambda i: (0, i))],
    out_specs=pl.BlockSpec((QROWS, PROW), lambda i: (i, 0)),
    out_shape=jax.ShapeDtypeStruct((PACKED_ROWS, PROW), jnp.float32),
)

_mesh = plsc.VectorSubcoreMesh(core_axis_name="c", subcore_axis_name="s")

_QSHIFT = TBLK.bit_length() - 1
_QBITS = QROWS.bit_length() - 1
_AMASK = PACK - 1


@functools.partial(
    pl.kernel,
    mesh=_mesh,
    out_type=jax.ShapeDtypeStruct((BATCH,), jnp.float32),
    scratch_types=[
        pltpu.VMEM((B_PER_W, 2), jnp.int32),
        pltpu.VMEM((NCHUNK, CHUNK), jnp.int32),
        pltpu.VMEM((NCHUNK, CHUNK), jnp.int32),
        pltpu.VMEM((B_PER_W, PROW), jnp.float32),
        pltpu.VMEM((EMBED_DIM, B_PER_W), jnp.float32),
        pltpu.VMEM((B_PER_W,), jnp.float32),
        pltpu.VMEM((1,), jnp.float32),
        pltpu.SemaphoreType.DMA,
        pltpu.SemaphoreType.DMA,
    ],
    compiler_params=pltpu.CompilerParams(
        needs_layout_passes=False, use_tc_tiling_on_sc=False),
)
def _mf_kernel(pairs_hbm, utab_hbm, itab_hbm, bias_hbm, out_hbm,
               pairs_v, uq_v, iq_v, rows_v, uemb_v, out_v, bias_v,
               sem_u, sem_i):
    wid = lax.axis_index("s") * 2 + lax.axis_index("c")
    base = wid * B_PER_W

    pltpu.sync_copy(pairs_hbm.at[pl.ds(base, B_PER_W)], pairs_v)
    pltpu.sync_copy(bias_hbm, bias_v)

    iota = lax.iota(jnp.int32, 16)
    zeros16 = jnp.zeros((16,), jnp.int32)
    ones16 = jnp.ones((16,), jnp.int32)

    def _qid(vals):
        return jnp.bitwise_or(
            lax.shift_left(lax.shift_right_logical(vals, _QSHIFT), _QBITS),
            jnp.bitwise_and(vals, QROWS - 1))

    for g in range(GROUPS):
        rows = g * 16 + iota
        c, off = divmod(g * 16, CHUNK)
        uq_v[c, pl.ds(off, 16)] = _qid(plsc.load_gather(pairs_v, [rows, zeros16]))
        iq_v[c, pl.ds(off, 16)] = _qid(plsc.load_gather(pairs_v, [rows, ones16]))

    def _gather_rows(tab_hbm, q_v, sem):
        copies = []
        for c in range(NCHUNK):
            copies.append(pltpu.make_async_copy(
                tab_hbm.at[q_v.at[c]],
                rows_v.at[pl.ds(c * CHUNK, CHUNK)], sem))
        for cp in copies:
            cp.start()
        for cp in copies:
            cp.wait()

    def _lane_base(vals):
        return lax.shift_left(
            jnp.bitwise_and(lax.shift_right_logical(vals, _QBITS), _AMASK), 4)

    _gather_rows(utab_hbm, uq_v, sem_u)

    def extract_u(g, carry):
        rows = g * 16 + iota
        lbu = _lane_base(plsc.load_gather(pairs_v, [rows, zeros16]))
        for d in range(EMBED_DIM):
            uemb_v[d, pl.ds(g * 16, 16)] = plsc.load_gather(
                rows_v, [rows, lbu + d])
        return carry

    lax.fori_loop(0, GROUPS, extract_u, 0)

    _gather_rows(itab_hbm, iq_v, sem_i)

    bias_vec = plsc.load_gather(bias_v, [zeros16])

    def dot_body(g, carry):
        rows = g * 16 + iota
        sl = pl.ds(g * 16, 16)
        lbi = _lane_base(plsc.load_gather(pairs_v, [rows, ones16]))
        acc = bias_vec
        for d in range(EMBED_DIM):
            v = plsc.load_gather(rows_v, [rows, lbi + d])
            acc = acc + uemb_v[d, sl] * v
        out_v[sl] = acc
        return carry

    lax.fori_loop(0, GROUPS, dot_body, 0)

    pltpu.sync_copy(out_v, out_hbm.at[pl.ds(base, B_PER_W)])


def kernel(sparse_inputs, user_table, item_table, bias):
    pairs = sparse_inputs.astype(jnp.int32)
    ut_p = _pack_table(user_table.T)
    it_p = _pack_table(item_table.T)
    return _mf_kernel(pairs, ut_p, it_p, bias)


# speedup vs baseline: 1.7534x; 1.1629x over previous
import functools

import jax
import jax.numpy as jnp
from jax import lax
from jax.experimental import pallas as pl
from jax.experimental.pallas import tpu as pltpu
from jax.experimental.pallas import tpu_sc as plsc

VOCAB1 = 1000001
BATCH = 16384
EMBED_DIM = 16
PACK = 8
NUM_WORKERS = 32
B_PER_W = BATCH // NUM_WORKERS
CHUNK = 128
NCHUNK = B_PER_W // CHUNK
GROUPS = B_PER_W // 16

TBLK = 16384
NBLK = (VOCAB1 + TBLK - 1) // TBLK
QROWS = TBLK // PACK
PACKED_ROWS = NBLK * QROWS
PROW = PACK * EMBED_DIM


def _pack_block(x):
    eye = jnp.eye(EMBED_DIM, dtype=jnp.float32)
    acc = None
    for a in range(PACK):
        ea = jnp.pad(
            eye, ((0, 0), (a * EMBED_DIM, (PACK - 1 - a) * EMBED_DIM)))
        part = jax.lax.dot_general(
            x[:, a * QROWS:(a + 1) * QROWS], ea, (((0,), (0,)), ((), ())),
            preferred_element_type=jnp.float32)
        acc = part if acc is None else acc + part
    return acc


def _pack2_body(xu_ref, xi_ref, ou_ref, oi_ref):
    ou_ref[...] = _pack_block(xu_ref[...])
    oi_ref[...] = _pack_block(xi_ref[...])


_in_spec = pl.BlockSpec((EMBED_DIM, TBLK), lambda i: (0, i))
_out_spec = pl.BlockSpec((QROWS, PROW), lambda i: (i, 0))
_out_type = jax.ShapeDtypeStruct((PACKED_ROWS, PROW), jnp.float32)

_pack_tables = pl.pallas_call(
    _pack2_body,
    grid=(NBLK,),
    in_specs=[_in_spec, _in_spec],
    out_specs=[_out_spec, _out_spec],
    out_shape=[_out_type, _out_type],
)

_mesh = plsc.VectorSubcoreMesh(core_axis_name="c", subcore_axis_name="s")

_QSHIFT = TBLK.bit_length() - 1
_QBITS = QROWS.bit_length() - 1
_AMASK = PACK - 1


@functools.partial(
    pl.kernel,
    mesh=_mesh,
    out_type=jax.ShapeDtypeStruct((BATCH,), jnp.float32),
    scratch_types=[
        pltpu.VMEM((B_PER_W, 2), jnp.int32),
        pltpu.VMEM((NCHUNK, CHUNK), jnp.int32),
        pltpu.VMEM((NCHUNK, CHUNK), jnp.int32),
        pltpu.VMEM((B_PER_W, PROW), jnp.float32),
        pltpu.VMEM((EMBED_DIM, B_PER_W), jnp.float32),
        pltpu.VMEM((B_PER_W,), jnp.float32),
        pltpu.VMEM((1,), jnp.float32),
        pltpu.SemaphoreType.DMA,
        pltpu.SemaphoreType.DMA,
    ],
    compiler_params=pltpu.CompilerParams(
        needs_layout_passes=False, use_tc_tiling_on_sc=False),
)
def _mf_kernel(pairs_hbm, utab_hbm, itab_hbm, bias_hbm, out_hbm,
               pairs_v, uq_v, iq_v, rows_v, uemb_v, out_v, bias_v,
               sem_u, sem_i):
    wid = lax.axis_index("s") * 2 + lax.axis_index("c")
    base = wid * B_PER_W

    pltpu.sync_copy(pairs_hbm.at[pl.ds(base, B_PER_W)], pairs_v)
    pltpu.sync_copy(bias_hbm, bias_v)

    iota = lax.iota(jnp.int32, 16)
    zeros16 = jnp.zeros((16,), jnp.int32)
    ones16 = jnp.ones((16,), jnp.int32)

    def _qid(vals):
        return jnp.bitwise_or(
            lax.shift_left(lax.shift_right_logical(vals, _QSHIFT), _QBITS),
            jnp.bitwise_and(vals, QROWS - 1))

    for g in range(GROUPS):
        rows = g * 16 + iota
        c, off = divmod(g * 16, CHUNK)
        uq_v[c, pl.ds(off, 16)] = _qid(plsc.load_gather(pairs_v, [rows, zeros16]))
        iq_v[c, pl.ds(off, 16)] = _qid(plsc.load_gather(pairs_v, [rows, ones16]))

    def _gather_rows(tab_hbm, q_v, sem):
        copies = []
        for c in range(NCHUNK):
            copies.append(pltpu.make_async_copy(
                tab_hbm.at[q_v.at[c]],
                rows_v.at[pl.ds(c * CHUNK, CHUNK)], sem))
        for cp in copies:
            cp.start()
        for cp in copies:
            cp.wait()

    def _lane_base(vals):
        return lax.shift_left(
            jnp.bitwise_and(lax.shift_right_logical(vals, _QBITS), _AMASK), 4)

    _gather_rows(utab_hbm, uq_v, sem_u)

    def extract_u(g, carry):
        rows = g * 16 + iota
        lbu = _lane_base(plsc.load_gather(pairs_v, [rows, zeros16]))
        for d in range(EMBED_DIM):
            uemb_v[d, pl.ds(g * 16, 16)] = plsc.load_gather(
                rows_v, [rows, lbu + d])
        return carry

    lax.fori_loop(0, GROUPS, extract_u, 0)

    _gather_rows(itab_hbm, iq_v, sem_i)

    bias_vec = plsc.load_gather(bias_v, [zeros16])

    def dot_body(g, carry):
        rows = g * 16 + iota
        sl = pl.ds(g * 16, 16)
        lbi = _lane_base(plsc.load_gather(pairs_v, [rows, ones16]))
        acc = bias_vec
        for d in range(EMBED_DIM):
            v = plsc.load_gather(rows_v, [rows, lbi + d])
            acc = acc + uemb_v[d, sl] * v
        out_v[sl] = acc
        return carry

    lax.fori_loop(0, GROUPS, dot_body, 0)

    pltpu.sync_copy(out_v, out_hbm.at[pl.ds(base, B_PER_W)])


def kernel(sparse_inputs, user_table, item_table, bias):
    pairs = sparse_inputs.astype(jnp.int32)
    ut_p, it_p = _pack_tables(user_table.T, item_table.T)
    return _mf_kernel(pairs, ut_p, it_p, bias)


# speedup vs baseline: 3.9653x; 2.2615x over previous
import functools

import jax
import jax.numpy as jnp
from jax import lax
from jax.experimental import pallas as pl
from jax.experimental.pallas import tpu as pltpu
from jax.experimental.pallas import tpu_sc as plsc

VOCAB1 = 1000001
BATCH = 16384
EMBED_DIM = 16
PACK = 8
NUM_WORKERS = 32
B_PER_W = BATCH // NUM_WORKERS
CHUNK = 128
NCHUNK = B_PER_W // CHUNK
GROUPS = B_PER_W // 16

TBLK = 16384
NBLK = (VOCAB1 + TBLK - 1) // TBLK
QROWS = TBLK // PACK
PACKED_ROWS = NBLK * QROWS
PROW = PACK * EMBED_DIM


def _pack_block(x):
    x8 = jnp.concatenate(
        [x[:, a * QROWS:(a + 1) * QROWS] for a in range(PACK)], axis=0)
    eye = jnp.eye(PROW, dtype=jnp.float32)
    return jax.lax.dot_general(
        x8, eye, (((0,), (0,)), ((), ())),
        preferred_element_type=jnp.float32)


def _pack2_body(xu_ref, xi_ref, ou_ref, oi_ref):
    ou_ref[...] = _pack_block(xu_ref[...])
    oi_ref[...] = _pack_block(xi_ref[...])


_in_spec = pl.BlockSpec((EMBED_DIM, TBLK), lambda i: (0, i))
_out_spec = pl.BlockSpec((QROWS, PROW), lambda i: (i, 0))
_out_type = jax.ShapeDtypeStruct((PACKED_ROWS, PROW), jnp.float32)

_pack_tables = pl.pallas_call(
    _pack2_body,
    grid=(NBLK,),
    in_specs=[_in_spec, _in_spec],
    out_specs=[_out_spec, _out_spec],
    out_shape=[_out_type, _out_type],
)

_mesh = plsc.VectorSubcoreMesh(core_axis_name="c", subcore_axis_name="s")

_QSHIFT = TBLK.bit_length() - 1
_QBITS = QROWS.bit_length() - 1
_AMASK = PACK - 1


@functools.partial(
    pl.kernel,
    mesh=_mesh,
    out_type=jax.ShapeDtypeStruct((BATCH,), jnp.float32),
    scratch_types=[
        pltpu.VMEM((B_PER_W, 2), jnp.int32),
        pltpu.VMEM((NCHUNK, CHUNK), jnp.int32),
        pltpu.VMEM((NCHUNK, CHUNK), jnp.int32),
        pltpu.VMEM((B_PER_W, PROW), jnp.float32),
        pltpu.VMEM((EMBED_DIM, B_PER_W), jnp.float32),
        pltpu.VMEM((B_PER_W,), jnp.float32),
        pltpu.VMEM((1,), jnp.float32),
        pltpu.SemaphoreType.DMA,
        pltpu.SemaphoreType.DMA,
    ],
    compiler_params=pltpu.CompilerParams(
        needs_layout_passes=False, use_tc_tiling_on_sc=False),
)
def _mf_kernel(pairs_hbm, utab_hbm, itab_hbm, bias_hbm, out_hbm,
               pairs_v, uq_v, iq_v, rows_v, uemb_v, out_v, bias_v,
               sem_u, sem_i):
    wid = lax.axis_index("s") * 2 + lax.axis_index("c")
    base = wid * B_PER_W

    pltpu.sync_copy(pairs_hbm.at[pl.ds(base, B_PER_W)], pairs_v)
    pltpu.sync_copy(bias_hbm, bias_v)

    iota = lax.iota(jnp.int32, 16)
    zeros16 = jnp.zeros((16,), jnp.int32)
    ones16 = jnp.ones((16,), jnp.int32)

    def _qid(vals):
        return jnp.bitwise_or(
            lax.shift_left(lax.shift_right_logical(vals, _QSHIFT), _QBITS),
            jnp.bitwise_and(vals, QROWS - 1))

    for g in range(GROUPS):
        rows = g * 16 + iota
        c, off = divmod(g * 16, CHUNK)
        uq_v[c, pl.ds(off, 16)] = _qid(plsc.load_gather(pairs_v, [rows, zeros16]))
        iq_v[c, pl.ds(off, 16)] = _qid(plsc.load_gather(pairs_v, [rows, ones16]))

    def _gather_rows(tab_hbm, q_v, sem):
        copies = []
        for c in range(NCHUNK):
            copies.append(pltpu.make_async_copy(
                tab_hbm.at[q_v.at[c]],
                rows_v.at[pl.ds(c * CHUNK, CHUNK)], sem))
        for cp in copies:
            cp.start()
        for cp in copies:
            cp.wait()

    def _lane_base(vals):
        return lax.shift_left(
            jnp.bitwise_and(lax.shift_right_logical(vals, _QBITS), _AMASK), 4)

    _gather_rows(utab_hbm, uq_v, sem_u)

    def extract_u(g, carry):
        rows = g * 16 + iota
        lbu = _lane_base(plsc.load_gather(pairs_v, [rows, zeros16]))
        for d in range(EMBED_DIM):
            uemb_v[d, pl.ds(g * 16, 16)] = plsc.load_gather(
                rows_v, [rows, lbu + d])
        return carry

    lax.fori_loop(0, GROUPS, extract_u, 0)

    _gather_rows(itab_hbm, iq_v, sem_i)

    bias_vec = plsc.load_gather(bias_v, [zeros16])

    def dot_body(g, carry):
        rows = g * 16 + iota
        sl = pl.ds(g * 16, 16)
        lbi = _lane_base(plsc.load_gather(pairs_v, [rows, ones16]))
        acc = bias_vec
        for d in range(EMBED_DIM):
            v = plsc.load_gather(rows_v, [rows, lbi + d])
            acc = acc + uemb_v[d, sl] * v
        out_v[sl] = acc
        return carry

    lax.fori_loop(0, GROUPS, dot_body, 0)

    pltpu.sync_copy(out_v, out_hbm.at[pl.ds(base, B_PER_W)])


def kernel(sparse_inputs, user_table, item_table, bias):
    pairs = sparse_inputs.astype(jnp.int32)
    ut_p, it_p = _pack_tables(user_table.T, item_table.T)
    return _mf_kernel(pairs, ut_p, it_p, bias)


# speedup vs baseline: 4.3931x; 1.1079x over previous
import functools

import jax
import jax.numpy as jnp
from jax import lax
from jax.experimental import pallas as pl
from jax.experimental.pallas import tpu as pltpu
from jax.experimental.pallas import tpu_sc as plsc

VOCAB1 = 1000001
BATCH = 16384
EMBED_DIM = 16
PACK = 8
NUM_WORKERS = 32
B_PER_W = BATCH // NUM_WORKERS
CHUNK = 128
NCHUNK = B_PER_W // CHUNK
GROUPS = B_PER_W // 16

TBLK = 32768
NBLK = (VOCAB1 + TBLK - 1) // TBLK
QROWS = TBLK // PACK
PACKED_ROWS = NBLK * QROWS
PROW = PACK * EMBED_DIM


def _pack_block(x):
    x8 = jnp.concatenate(
        [x[:, a * QROWS:(a + 1) * QROWS] for a in range(PACK)], axis=0)
    eye = jnp.eye(PROW, dtype=jnp.float32)
    return jax.lax.dot_general(
        x8, eye, (((0,), (0,)), ((), ())),
        preferred_element_type=jnp.float32)


def _pack2_body(xu_ref, xi_ref, ou_ref, oi_ref):
    ou_ref[...] = _pack_block(xu_ref[...])
    oi_ref[...] = _pack_block(xi_ref[...])


_in_spec = pl.BlockSpec((EMBED_DIM, TBLK), lambda i: (0, i))
_out_spec = pl.BlockSpec((QROWS, PROW), lambda i: (i, 0))
_out_type = jax.ShapeDtypeStruct((PACKED_ROWS, PROW), jnp.float32)

_pack_tables = pl.pallas_call(
    _pack2_body,
    grid=(NBLK,),
    in_specs=[_in_spec, _in_spec],
    out_specs=[_out_spec, _out_spec],
    out_shape=[_out_type, _out_type],
)

_mesh = plsc.VectorSubcoreMesh(core_axis_name="c", subcore_axis_name="s")

_QSHIFT = TBLK.bit_length() - 1
_QBITS = QROWS.bit_length() - 1
_AMASK = PACK - 1


@functools.partial(
    pl.kernel,
    mesh=_mesh,
    out_type=jax.ShapeDtypeStruct((BATCH,), jnp.float32),
    scratch_types=[
        pltpu.VMEM((B_PER_W, 2), jnp.int32),
        pltpu.VMEM((NCHUNK, CHUNK), jnp.int32),
        pltpu.VMEM((NCHUNK, CHUNK), jnp.int32),
        pltpu.VMEM((B_PER_W, PROW), jnp.float32),
        pltpu.VMEM((EMBED_DIM, B_PER_W), jnp.float32),
        pltpu.VMEM((B_PER_W,), jnp.float32),
        pltpu.VMEM((1,), jnp.float32),
        pltpu.SemaphoreType.DMA,
        pltpu.SemaphoreType.DMA,
    ],
    compiler_params=pltpu.CompilerParams(
        needs_layout_passes=False, use_tc_tiling_on_sc=False),
)
def _mf_kernel(pairs_hbm, utab_hbm, itab_hbm, bias_hbm, out_hbm,
               pairs_v, uq_v, iq_v, rows_v, uemb_v, out_v, bias_v,
               sem_u, sem_i):
    wid = lax.axis_index("s") * 2 + lax.axis_index("c")
    base = wid * B_PER_W

    pltpu.sync_copy(pairs_hbm.at[pl.ds(base, B_PER_W)], pairs_v)
    pltpu.sync_copy(bias_hbm, bias_v)

    iota = lax.iota(jnp.int32, 16)
    zeros16 = jnp.zeros((16,), jnp.int32)
    ones16 = jnp.ones((16,), jnp.int32)

    def _qid(vals):
        return jnp.bitwise_or(
            lax.shift_left(lax.shift_right_logical(vals, _QSHIFT), _QBITS),
            jnp.bitwise_and(vals, QROWS - 1))

    for g in range(GROUPS):
        rows = g * 16 + iota
        c, off = divmod(g * 16, CHUNK)
        uq_v[c, pl.ds(off, 16)] = _qid(plsc.load_gather(pairs_v, [rows, zeros16]))
        iq_v[c, pl.ds(off, 16)] = _qid(plsc.load_gather(pairs_v, [rows, ones16]))

    def _gather_rows(tab_hbm, q_v, sem):
        copies = []
        for c in range(NCHUNK):
            copies.append(pltpu.make_async_copy(
                tab_hbm.at[q_v.at[c]],
                rows_v.at[pl.ds(c * CHUNK, CHUNK)], sem))
        for cp in copies:
            cp.start()
        for cp in copies:
            cp.wait()

    def _lane_base(vals):
        return lax.shift_left(
            jnp.bitwise_and(lax.shift_right_logical(vals, _QBITS), _AMASK), 4)

    _gather_rows(utab_hbm, uq_v, sem_u)

    def extract_u(g, carry):
        rows = g * 16 + iota
        lbu = _lane_base(plsc.load_gather(pairs_v, [rows, zeros16]))
        for d in range(EMBED_DIM):
            uemb_v[d, pl.ds(g * 16, 16)] = plsc.load_gather(
                rows_v, [rows, lbu + d])
        return carry

    lax.fori_loop(0, GROUPS, extract_u, 0)

    _gather_rows(itab_hbm, iq_v, sem_i)

    bias_vec = plsc.load_gather(bias_v, [zeros16])

    def dot_body(g, carry):
        rows = g * 16 + iota
        sl = pl.ds(g * 16, 16)
        lbi = _lane_base(plsc.load_gather(pairs_v, [rows, ones16]))
        acc = bias_vec
        for d in range(EMBED_DIM):
            v = plsc.load_gather(rows_v, [rows, lbi + d])
            acc = acc + uemb_v[d, sl] * v
        out_v[sl] = acc
        return carry

    lax.fori_loop(0, GROUPS, dot_body, 0)

    pltpu.sync_copy(out_v, out_hbm.at[pl.ds(base, B_PER_W)])


def kernel(sparse_inputs, user_table, item_table, bias):
    pairs = sparse_inputs.astype(jnp.int32)
    ut_p, it_p = _pack_tables(user_table.T, item_table.T)
    return _mf_kernel(pairs, ut_p, it_p, bias)


# speedup vs baseline: 4.4239x; 1.0070x over previous
import functools

import jax
import jax.numpy as jnp
from jax import lax
from jax.experimental import pallas as pl
from jax.experimental.pallas import tpu as pltpu
from jax.experimental.pallas import tpu_sc as plsc

VOCAB1 = 1000001
BATCH = 16384
EMBED_DIM = 16
PACK = 8
NUM_WORKERS = 32
B_PER_W = BATCH // NUM_WORKERS
CHUNK = 128
NCHUNK = B_PER_W // CHUNK
GROUPS = B_PER_W // 16

TBLK = 65536
NBLK = (VOCAB1 + TBLK - 1) // TBLK
QROWS = TBLK // PACK
PACKED_ROWS = NBLK * QROWS
PROW = PACK * EMBED_DIM


def _pack_block(x):
    x8 = jnp.concatenate(
        [x[:, a * QROWS:(a + 1) * QROWS] for a in range(PACK)], axis=0)
    eye = jnp.eye(PROW, dtype=jnp.float32)
    return jax.lax.dot_general(
        x8, eye, (((0,), (0,)), ((), ())),
        preferred_element_type=jnp.float32)


def _pack2_body(xu_ref, xi_ref, ou_ref, oi_ref):
    ou_ref[...] = _pack_block(xu_ref[...])
    oi_ref[...] = _pack_block(xi_ref[...])


_in_spec = pl.BlockSpec((EMBED_DIM, TBLK), lambda i: (0, i))
_out_spec = pl.BlockSpec((QROWS, PROW), lambda i: (i, 0))
_out_type = jax.ShapeDtypeStruct((PACKED_ROWS, PROW), jnp.float32)

_pack_tables = pl.pallas_call(
    _pack2_body,
    grid=(NBLK,),
    in_specs=[_in_spec, _in_spec],
    out_specs=[_out_spec, _out_spec],
    out_shape=[_out_type, _out_type],
)

_mesh = plsc.VectorSubcoreMesh(core_axis_name="c", subcore_axis_name="s")

_QSHIFT = TBLK.bit_length() - 1
_QBITS = QROWS.bit_length() - 1
_AMASK = PACK - 1


@functools.partial(
    pl.kernel,
    mesh=_mesh,
    out_type=jax.ShapeDtypeStruct((BATCH,), jnp.float32),
    scratch_types=[
        pltpu.VMEM((B_PER_W, 2), jnp.int32),
        pltpu.VMEM((NCHUNK, CHUNK), jnp.int32),
        pltpu.VMEM((NCHUNK, CHUNK), jnp.int32),
        pltpu.VMEM((B_PER_W, PROW), jnp.float32),
        pltpu.VMEM((EMBED_DIM, B_PER_W), jnp.float32),
        pltpu.VMEM((B_PER_W,), jnp.float32),
        pltpu.VMEM((1,), jnp.float32),
        pltpu.SemaphoreType.DMA,
        pltpu.SemaphoreType.DMA,
    ],
    compiler_params=pltpu.CompilerParams(
        needs_layout_passes=False, use_tc_tiling_on_sc=False),
)
def _mf_kernel(pairs_hbm, utab_hbm, itab_hbm, bias_hbm, out_hbm,
               pairs_v, uq_v, iq_v, rows_v, uemb_v, out_v, bias_v,
               sem_u, sem_i):
    wid = lax.axis_index("s") * 2 + lax.axis_index("c")
    base = wid * B_PER_W

    pltpu.sync_copy(pairs_hbm.at[pl.ds(base, B_PER_W)], pairs_v)
    pltpu.sync_copy(bias_hbm, bias_v)

    iota = lax.iota(jnp.int32, 16)
    zeros16 = jnp.zeros((16,), jnp.int32)
    ones16 = jnp.ones((16,), jnp.int32)

    def _qid(vals):
        return jnp.bitwise_or(
            lax.shift_left(lax.shift_right_logical(vals, _QSHIFT), _QBITS),
            jnp.bitwise_and(vals, QROWS - 1))

    for g in range(GROUPS):
        rows = g * 16 + iota
        c, off = divmod(g * 16, CHUNK)
        uq_v[c, pl.ds(off, 16)] = _qid(plsc.load_gather(pairs_v, [rows, zeros16]))
        iq_v[c, pl.ds(off, 16)] = _qid(plsc.load_gather(pairs_v, [rows, ones16]))

    def _gather_rows(tab_hbm, q_v, sem):
        copies = []
        for c in range(NCHUNK):
            copies.append(pltpu.make_async_copy(
                tab_hbm.at[q_v.at[c]],
                rows_v.at[pl.ds(c * CHUNK, CHUNK)], sem))
        for cp in copies:
            cp.start()
        for cp in copies:
            cp.wait()

    def _lane_base(vals):
        return lax.shift_left(
            jnp.bitwise_and(lax.shift_right_logical(vals, _QBITS), _AMASK), 4)

    _gather_rows(utab_hbm, uq_v, sem_u)

    def extract_u(g, carry):
        rows = g * 16 + iota
        lbu = _lane_base(plsc.load_gather(pairs_v, [rows, zeros16]))
        for d in range(EMBED_DIM):
            uemb_v[d, pl.ds(g * 16, 16)] = plsc.load_gather(
                rows_v, [rows, lbu + d])
        return carry

    lax.fori_loop(0, GROUPS, extract_u, 0)

    _gather_rows(itab_hbm, iq_v, sem_i)

    bias_vec = plsc.load_gather(bias_v, [zeros16])

    def dot_body(g, carry):
        rows = g * 16 + iota
        sl = pl.ds(g * 16, 16)
        lbi = _lane_base(plsc.load_gather(pairs_v, [rows, ones16]))
        acc = bias_vec
        for d in range(EMBED_DIM):
            v = plsc.load_gather(rows_v, [rows, lbi + d])
            acc = acc + uemb_v[d, sl] * v
        out_v[sl] = acc
        return carry

    lax.fori_loop(0, GROUPS, dot_body, 0)

    pltpu.sync_copy(out_v, out_hbm.at[pl.ds(base, B_PER_W)])


def kernel(sparse_inputs, user_table, item_table, bias):
    pairs = sparse_inputs.astype(jnp.int32)
    ut_p, it_p = _pack_tables(user_table.T, item_table.T)
    return _mf_kernel(pairs, ut_p, it_p, bias)
